# baseline (speedup 1.0000x reference)
import functools

import jax
import jax.numpy as jnp
from jax import lax
from jax.experimental import pallas as pl
from jax.experimental.pallas import tpu as pltpu
from jax.experimental.pallas import tpu_sc as plsc

B = 16384
K = 32
GROUP = 128 // K
N_GROUPS = 1000000 // GROUP
LANES = 16
NUM_CORES = 2
NUM_SUBCORES = 16
NUM_WORKERS = NUM_CORES * NUM_SUBCORES
BPW = B // NUM_WORKERS
IDX_CHUNK = 128
N_CHUNKS = BPW // IDX_CHUNK


def _mf_kernel(user_hbm, item_hbm, uf_hbm, if_hbm, out_hbm,
               uidx_v, iidx_v, ugrp_v, igrp_v, urows_v, irows_v, out_v, sem):
    wid = lax.axis_index("s") * NUM_CORES + lax.axis_index("c")
    base = wid * BPW

    for j in range(N_CHUNKS):
        off = base + j * IDX_CHUNK
        pltpu.sync_copy(user_hbm.at[pl.ds(off, IDX_CHUNK)], uidx_v.at[j])
        pltpu.sync_copy(item_hbm.at[pl.ds(off, IDX_CHUNK)], iidx_v.at[j])
    for j in range(N_CHUNKS):
        for s in range(IDX_CHUNK // LANES):
            sl = pl.ds(s * LANES, LANES)
            u = uidx_v[j, sl]
            i = iidx_v[j, sl]
            ugrp_v[j, sl] = (
                jax.lax.shift_left(jax.lax.shift_right_logical(u, 11), 9)
                | (u & 511))
            igrp_v[j, sl] = (
                jax.lax.shift_left(jax.lax.shift_right_logical(i, 11), 9)
                | (i & 511))

    def fire(j):
        buf = j % 2
        return (pltpu.async_copy(uf_hbm.at[ugrp_v.at[j]], urows_v.at[buf], sem),
                pltpu.async_copy(if_hbm.at[igrp_v.at[j]], irows_v.at[buf], sem))

    lanes = lax.iota(jnp.int32, LANES)
    pending = fire(0)

    for j in range(N_CHUNKS):
        for c in pending:
            c.wait()
        if j + 1 < N_CHUNKS:
            pending = fire(j + 1)
        buf = j % 2
        for g in range(IDX_CHUNK // LANES):
            sl = pl.ds(g * LANES, LANES)
            coff = jax.lax.shift_left(
                jax.lax.shift_right_logical(uidx_v[j, sl], 9) & 3, 5)
            coff_i = jax.lax.shift_left(
                jax.lax.shift_right_logical(iidx_v[j, sl], 9) & 3, 5)
            rows16 = g * LANES + lanes
            acc = jnp.zeros((LANES,), jnp.float32)
            for k in range(K):
                u = plsc.load_gather(urows_v.at[buf], [rows16, coff + k])
                v = plsc.load_gather(irows_v.at[buf], [rows16, coff_i + k])
                acc = acc + u * v
            out_v[pl.ds(j * IDX_CHUNK + g * LANES, LANES)] = acc

    pltpu.sync_copy(out_v, out_hbm.at[pl.ds(base, BPW)])


TR_SUB = 512
TR_GRID = (1000000 + 4 * TR_SUB - 1) // (4 * TR_SUB)
V_ROWS = TR_GRID * TR_SUB
LAST_SUB = (1000000 + TR_SUB - 1) // TR_SUB - 1


def _tr_kernel(x0, x1, x2, x3, o_ref):
    ident = jnp.eye(K, dtype=jnp.float32)
    for i, x in enumerate((x0, x1, x2, x3)):
        o_ref[:, K * i:K * (i + 1)] = jax.lax.dot_general(
            x[...], ident, (((0,), (0,)), ((), ())),
            preferred_element_type=jnp.float32)


def _stage_table(table):
    tr = pl.pallas_call(
        _tr_kernel,
        grid=(TR_GRID,),
        in_specs=[pl.BlockSpec(
            (K, TR_SUB),
            lambda c, i=i: (0, jnp.minimum(4 * c + i, LAST_SUB)))
                  for i in range(4)],
        out_specs=pl.BlockSpec((TR_SUB, 128), lambda c: (c, 0)),
        out_shape=jax.ShapeDtypeStruct((V_ROWS, 128), jnp.float32),
    )
    return tr(table.T, table.T, table.T, table.T)


@jax.jit
def kernel(user, item, user_factors, item_factors):
    mesh = plsc.VectorSubcoreMesh(core_axis_name="c", subcore_axis_name="s")
    run = functools.partial(
        pl.kernel,
        out_type=jax.ShapeDtypeStruct((B,), jnp.float32),
        mesh=mesh,
        compiler_params=pltpu.CompilerParams(needs_layout_passes=False,
                                             use_tc_tiling_on_sc=False),
        scratch_types=[
            pltpu.VMEM((N_CHUNKS, IDX_CHUNK), jnp.int32),
            pltpu.VMEM((N_CHUNKS, IDX_CHUNK), jnp.int32),
            pltpu.VMEM((N_CHUNKS, IDX_CHUNK), jnp.int32),
            pltpu.VMEM((N_CHUNKS, IDX_CHUNK), jnp.int32),
            pltpu.VMEM((2, IDX_CHUNK, 128), jnp.float32),
            pltpu.VMEM((2, IDX_CHUNK, 128), jnp.float32),
            pltpu.VMEM((BPW,), jnp.float32),
            pltpu.SemaphoreType.DMA,
        ],
    )(_mf_kernel)
    return run(user, item, _stage_table(user_factors),
               _stage_table(item_factors))


# speedup vs baseline: 1.6273x; 1.6273x over previous
import functools

import jax
import jax.numpy as jnp
from jax import lax
from jax.experimental import pallas as pl
from jax.experimental.pallas import tpu as pltpu
from jax.experimental.pallas import tpu_sc as plsc

B = 16384
K = 32
GROUP = 128 // K
N_GROUPS = 1000000 // GROUP
LANES = 16
NUM_CORES = 2
NUM_SUBCORES = 16
NUM_WORKERS = NUM_CORES * NUM_SUBCORES
BPW = B // NUM_WORKERS
IDX_CHUNK = 128
N_CHUNKS = BPW // IDX_CHUNK


def _mf_kernel(user_hbm, item_hbm, uf_hbm, if_hbm, out_hbm,
               uidx_v, iidx_v, ugrp_v, igrp_v, urows_v, irows_v, out_v, sem):
    wid = lax.axis_index("s") * NUM_CORES + lax.axis_index("c")
    base = wid * BPW

    for j in range(N_CHUNKS):
        off = base + j * IDX_CHUNK
        pltpu.sync_copy(user_hbm.at[pl.ds(off, IDX_CHUNK)], uidx_v.at[j])
        pltpu.sync_copy(item_hbm.at[pl.ds(off, IDX_CHUNK)], iidx_v.at[j])
    for j in range(N_CHUNKS):
        for s in range(IDX_CHUNK // LANES):
            sl = pl.ds(s * LANES, LANES)
            u = uidx_v[j, sl]
            i = iidx_v[j, sl]
            ugrp_v[j, sl] = (
                jax.lax.shift_left(jax.lax.shift_right_logical(u, 11), 9)
                | (u & 511))
            igrp_v[j, sl] = (
                jax.lax.shift_left(jax.lax.shift_right_logical(i, 11), 9)
                | (i & 511))

    def fire(j):
        buf = j % 2
        return (pltpu.async_copy(uf_hbm.at[ugrp_v.at[j]], urows_v.at[buf], sem),
                pltpu.async_copy(if_hbm.at[igrp_v.at[j]], irows_v.at[buf], sem))

    lanes = lax.iota(jnp.int32, LANES)
    pending = fire(0)

    for j in range(N_CHUNKS):
        for c in pending:
            c.wait()
        if j + 1 < N_CHUNKS:
            pending = fire(j + 1)
        buf = j % 2
        for g in range(IDX_CHUNK // LANES):
            sl = pl.ds(g * LANES, LANES)
            coff = jax.lax.shift_left(
                jax.lax.shift_right_logical(uidx_v[j, sl], 9) & 3, 5)
            coff_i = jax.lax.shift_left(
                jax.lax.shift_right_logical(iidx_v[j, sl], 9) & 3, 5)
            rows16 = g * LANES + lanes
            acc = jnp.zeros((LANES,), jnp.float32)
            for k in range(K):
                u = plsc.load_gather(urows_v.at[buf], [rows16, coff + k])
                v = plsc.load_gather(irows_v.at[buf], [rows16, coff_i + k])
                acc = acc + u * v
            out_v[pl.ds(j * IDX_CHUNK + g * LANES, LANES)] = acc

    pltpu.sync_copy(out_v, out_hbm.at[pl.ds(base, BPW)])


TR_SUB = 512
TR_NSUB = 16
TR_COLS = TR_SUB * TR_NSUB
TR_GRID = (1000000 + TR_COLS - 1) // TR_COLS
V_ROWS = TR_GRID * TR_NSUB // 4 * TR_SUB
LAST_SUB = (1000000 + TR_SUB - 1) // TR_SUB - 1


def _tr_kernel(x_ref, o_ref):
    for s in range(TR_NSUB):
        g, i = divmod(s, 4)
        x = x_ref[:, pl.ds(s * TR_SUB, TR_SUB)]
        o_ref[pl.ds(g * TR_SUB, TR_SUB), K * i:K * (i + 1)] = (
            jnp.transpose(x, (1, 0)))


def _stage_table(table):
    tr = pl.pallas_call(
        _tr_kernel,
        grid=(TR_GRID,),
        in_specs=[pl.BlockSpec(
            (K, TR_COLS),
            lambda c: (0, jnp.minimum(c, (1000000 - 1) // TR_COLS)))],
        out_specs=pl.BlockSpec((TR_NSUB // 4 * TR_SUB, 128),
                               lambda c: (c, 0)),
        out_shape=jax.ShapeDtypeStruct((V_ROWS, 128), jnp.float32),
    )
    return tr(table.T)


@jax.jit
def kernel(user, item, user_factors, item_factors):
    mesh = plsc.VectorSubcoreMesh(core_axis_name="c", subcore_axis_name="s")
    run = functools.partial(
        pl.kernel,
        out_type=jax.ShapeDtypeStruct((B,), jnp.float32),
        mesh=mesh,
        compiler_params=pltpu.CompilerParams(needs_layout_passes=False,
                                             use_tc_tiling_on_sc=False),
        scratch_types=[
            pltpu.VMEM((N_CHUNKS, IDX_CHUNK), jnp.int32),
            pltpu.VMEM((N_CHUNKS, IDX_CHUNK), jnp.int32),
            pltpu.VMEM((N_CHUNKS, IDX_CHUNK), jnp.int32),
            pltpu.VMEM((N_CHUNKS, IDX_CHUNK), jnp.int32),
            pltpu.VMEM((2, IDX_CHUNK, 128), jnp.float32),
            pltpu.VMEM((2, IDX_CHUNK, 128), jnp.float32),
            pltpu.VMEM((BPW,), jnp.float32),
            pltpu.SemaphoreType.DMA,
        ],
    )(_mf_kernel)
    return run(user, item, _stage_table(user_factors),
               _stage_table(item_factors))


# speedup vs baseline: 1.6611x; 1.0208x over previous
import functools

import jax
import jax.numpy as jnp
from jax import lax
from jax.experimental import pallas as pl
from jax.experimental.pallas import tpu as pltpu
from jax.experimental.pallas import tpu_sc as plsc

B = 16384
K = 32
GROUP = 128 // K
N_GROUPS = 1000000 // GROUP
LANES = 16
NUM_CORES = 2
NUM_SUBCORES = 16
NUM_WORKERS = NUM_CORES * NUM_SUBCORES
BPW = B // NUM_WORKERS
IDX_CHUNK = 128
N_CHUNKS = BPW // IDX_CHUNK


def _mf_kernel(user_hbm, item_hbm, uf_hbm, if_hbm, out_hbm,
               uidx_v, iidx_v, ugrp_v, igrp_v, urows_v, irows_v, out_v, sem):
    wid = lax.axis_index("s") * NUM_CORES + lax.axis_index("c")
    base = wid * BPW

    for j in range(N_CHUNKS):
        off = base + j * IDX_CHUNK
        pltpu.sync_copy(user_hbm.at[pl.ds(off, IDX_CHUNK)], uidx_v.at[j])
        pltpu.sync_copy(item_hbm.at[pl.ds(off, IDX_CHUNK)], iidx_v.at[j])
    for j in range(N_CHUNKS):
        for s in range(IDX_CHUNK // LANES):
            sl = pl.ds(s * LANES, LANES)
            u = uidx_v[j, sl]
            i = iidx_v[j, sl]
            ugrp_v[j, sl] = (
                jax.lax.shift_left(jax.lax.shift_right_logical(u, 11), 9)
                | (u & 511))
            igrp_v[j, sl] = (
                jax.lax.shift_left(jax.lax.shift_right_logical(i, 11), 9)
                | (i & 511))

    def fire(j):
        buf = j % 2
        return (pltpu.async_copy(uf_hbm.at[ugrp_v.at[j]], urows_v.at[buf], sem),
                pltpu.async_copy(if_hbm.at[igrp_v.at[j]], irows_v.at[buf], sem))

    lanes = lax.iota(jnp.int32, LANES)
    pending = fire(0)

    for j in range(N_CHUNKS):
        for c in pending:
            c.wait()
        if j + 1 < N_CHUNKS:
            pending = fire(j + 1)
        buf = j % 2
        for g in range(IDX_CHUNK // LANES):
            sl = pl.ds(g * LANES, LANES)
            coff = jax.lax.shift_left(
                jax.lax.shift_right_logical(uidx_v[j, sl], 9) & 3, 5)
            coff_i = jax.lax.shift_left(
                jax.lax.shift_right_logical(iidx_v[j, sl], 9) & 3, 5)
            rows16 = g * LANES + lanes
            acc = jnp.zeros((LANES,), jnp.float32)
            for k in range(K):
                u = plsc.load_gather(urows_v.at[buf], [rows16, coff + k])
                v = plsc.load_gather(irows_v.at[buf], [rows16, coff_i + k])
                acc = acc + u * v
            out_v[pl.ds(j * IDX_CHUNK + g * LANES, LANES)] = acc

    pltpu.sync_copy(out_v, out_hbm.at[pl.ds(base, BPW)])


TR_SUB = 512
TR_NSUB = 32
TR_COLS = TR_SUB * TR_NSUB
TR_GRID = (1000000 + TR_COLS - 1) // TR_COLS
V_ROWS = TR_GRID * TR_NSUB // 4 * TR_SUB
LAST_SUB = (1000000 + TR_SUB - 1) // TR_SUB - 1


def _tr_kernel(x_ref, o_ref):
    for s in range(TR_NSUB):
        g, i = divmod(s, 4)
        x = x_ref[:, pl.ds(s * TR_SUB, TR_SUB)]
        o_ref[pl.ds(g * TR_SUB, TR_SUB), K * i:K * (i + 1)] = (
            jnp.transpose(x, (1, 0)))


def _stage_table(table):
    tr = pl.pallas_call(
        _tr_kernel,
        grid=(TR_GRID,),
        in_specs=[pl.BlockSpec(
            (K, TR_COLS),
            lambda c: (0, jnp.minimum(c, (1000000 - 1) // TR_COLS)))],
        out_specs=pl.BlockSpec((TR_NSUB // 4 * TR_SUB, 128),
                               lambda c: (c, 0)),
        out_shape=jax.ShapeDtypeStruct((V_ROWS, 128), jnp.float32),
    )
    return tr(table.T)


@jax.jit
def kernel(user, item, user_factors, item_factors):
    mesh = plsc.VectorSubcoreMesh(core_axis_name="c", subcore_axis_name="s")
    run = functools.partial(
        pl.kernel,
        out_type=jax.ShapeDtypeStruct((B,), jnp.float32),
        mesh=mesh,
        compiler_params=pltpu.CompilerParams(needs_layout_passes=False,
                                             use_tc_tiling_on_sc=False),
        scratch_types=[
            pltpu.VMEM((N_CHUNKS, IDX_CHUNK), jnp.int32),
            pltpu.VMEM((N_CHUNKS, IDX_CHUNK), jnp.int32),
            pltpu.VMEM((N_CHUNKS, IDX_CHUNK), jnp.int32),
            pltpu.VMEM((N_CHUNKS, IDX_CHUNK), jnp.int32),
            pltpu.VMEM((2, IDX_CHUNK, 128), jnp.float32),
            pltpu.VMEM((2, IDX_CHUNK, 128), jnp.float32),
            pltpu.VMEM((BPW,), jnp.float32),
            pltpu.SemaphoreType.DMA,
        ],
    )(_mf_kernel)
    return run(user, item, _stage_table(user_factors),
               _stage_table(item_factors))
